# baseline (speedup 1.0000x reference)
import jax
import jax.numpy as jnp
from jax import lax
from jax.experimental import pallas as pl
from jax.experimental.pallas import tpu as pltpu
from jax.experimental.pallas import tpu_sc as plsc

N_NODES = 50000
N_PAD = 50048
N_DIM = 64
HALF = 32
N_LAYERS = 3
N_BATCH = 4096
N_PAIRS = 16384
TEMP = 0.5
LAMBDA_SSL = 1.0
LAMBDA_BPR = 1.0
LAMBDA_REG = 1e-4

NC = 2
NS = 16
SUB = 128
NSUB = 1
CHUNK = SUB * NSUB
NCHUNK = 396
NSETS = 6
NTRIP_IT = NCHUNK // NSETS
E_PAD = NS * NCHUNK * CHUNK
ROWS_PER_SUB = N_PAD // NS
ZROWS = 96
QUART = 16
TROWS = 3128

_SC_MESH = plsc.VectorSubcoreMesh(core_axis_name="c", subcore_axis_name="s",
                                  num_cores=NC, num_subcores=NS)
_SC_PARAMS = pltpu.CompilerParams(use_tc_tiling_on_sc=False)


def _spmm3_kernel(emb, src4, dst4, val4, out1, out2, out3, *refs):
  srcvs = refs[0:6]
  dstvs = refs[6:12]
  valvs = refs[12:18]
  rowss = refs[18:24]
  zbuf = refs[24]
  acc = refs[25]
  tblq = refs[26]
  semis = refs[27:33]
  semgs = refs[33:39]
  semss = refs[39:45]
  c = lax.axis_index("c")
  s = lax.axis_index("s")

  def _z(i, _):
    zv = jnp.zeros((16,), jnp.float32)
    zbuf[i, pl.ds(0, 16)] = zv
    return 0
  lax.fori_loop(0, ZROWS, _z, 0)

  def issue_i(g, x):
    pltpu.async_copy(src4.at[s, g], srcvs[x], semis[x])
    pltpu.async_copy(dst4.at[s, g], dstvs[x], semis[x])
    pltpu.async_copy(val4.at[s, g], valvs[x], semis[x])

  def wait_i(g, x):
    pltpu.make_async_copy(src4.at[s, g], srcvs[x], semis[x]).wait()
    pltpu.make_async_copy(dst4.at[s, g], dstvs[x], semis[x]).wait()
    pltpu.make_async_copy(val4.at[s, g], valvs[x], semis[x]).wait()

  def issue_g(x):
    pltpu.async_copy(tblq.at[srcvs[x].at[0]], rowss[x], semgs[x])

  def wait_g(x):
    pltpu.make_async_copy(tblq.at[srcvs[x].at[0]], rowss[x],
                          semgs[x]).wait()

  def issue_s(x):
    pltpu.async_copy(rowss[x], acc.at[dstvs[x].at[0]], semss[x], add=True)

  def wait_s(x):
    pltpu.make_async_copy(rowss[x], acc.at[dstvs[x].at[0]], semss[x]).wait()

  def do_scale(x):
    valv = valvs[x]
    rows = rowss[x]
    def _scale(e0, _):
      vv = valv[0, pl.ds(e0 * 16, 16)]
      for l in range(16):
        v = vv[l]
        rows[e0 * 16 + l, pl.ds(0, 16)] = (
            rows[e0 * 16 + l, pl.ds(0, 16)] * v)
      return 0
    lax.fori_loop(0, SUB // 16, _scale, 0)

  srcs = (emb, out1, out2)
  outs = (out1, out2, out3)

  t0 = TROWS * s

  for lyr in range(N_LAYERS):
    tbl = srcs[lyr]
    out = outs[lyr]

    for q in range(2):
      d0 = QUART * q
      pltpu.sync_copy(tbl.at[c, pl.ds(t0, TROWS), pl.ds(d0, QUART)],
                      tblq.at[pl.ds(t0, TROWS)])
      base = s * ROWS_PER_SUB
      for k in range(ROWS_PER_SUB // ZROWS):
        pltpu.sync_copy(zbuf, acc.at[pl.ds(base + k * ZROWS, ZROWS)])
      rem = ROWS_PER_SUB % ZROWS
      if rem:
        pltpu.sync_copy(zbuf.at[pl.ds(0, rem)],
                        acc.at[pl.ds(base + ROWS_PER_SUB - rem, rem)])
      plsc.subcore_barrier()

      issue_i(0, 0)
      issue_i(1, 1)
      issue_i(2, 2)
      wait_i(0, 0)
      issue_g(0)
      wait_i(1, 1)
      issue_g(1)

      def _trip(i, _):
        for x in range(NSETS):
          g = NSETS * i + x
          if x >= 3:
            wait_s((x + 3) % NSETS)
          else:
            @pl.when(i >= 1)
            def _():
              wait_s((x + 3) % NSETS)
          if x < 3:
            issue_i(g + 3, (x + 3) % NSETS)
          else:
            @pl.when(i < NTRIP_IT - 1)
            def _():
              issue_i(g + 3, (x + 3) % NSETS)
          if x < 4:
            wait_i(g + 2, (x + 2) % NSETS)
            issue_g((x + 2) % NSETS)
          else:
            @pl.when(i < NTRIP_IT - 1)
            def _():
              wait_i(g + 2, (x + 2) % NSETS)
              issue_g((x + 2) % NSETS)
          wait_g(x)
          do_scale(x)
          issue_s(x)
        return 0
      lax.fori_loop(0, NTRIP_IT, _trip, 0)
      wait_s(3)
      wait_s(4)
      wait_s(5)
      plsc.subcore_barrier()

      pltpu.sync_copy(acc.at[pl.ds(base, ROWS_PER_SUB)],
                      out.at[c, pl.ds(base, ROWS_PER_SUB), pl.ds(d0, QUART)])
      plsc.subcore_barrier()


@jax.jit
def _propagate3(emb_st, idx, vals):
  pad = E_PAD - idx.shape[1]
  src = jnp.concatenate([idx[1], jnp.zeros((pad,), jnp.int32)])
  dst = jnp.concatenate([idx[0], jnp.zeros((pad,), jnp.int32)])
  val = jnp.concatenate([vals, jnp.zeros((pad,), jnp.float32)])
  src4 = src.reshape(NS, NCHUNK, NSUB, SUB)
  dst4 = dst.reshape(NS, NCHUNK, NSUB, SUB)
  val4 = val.reshape(NS, NCHUNK, NSUB, SUB)

  f = pl.kernel(
      _spmm3_kernel,
      out_type=[jax.ShapeDtypeStruct((NC, N_PAD, HALF), jnp.float32)] * 3,
      mesh=_SC_MESH,
      scratch_types=(
          [pltpu.VMEM((NSUB, SUB), jnp.int32)] * 12
          + [pltpu.VMEM((NSUB, SUB), jnp.float32)] * 6
          + [pltpu.VMEM((SUB, QUART), jnp.float32)] * 6
          + [pltpu.VMEM((ZROWS, QUART), jnp.float32),
             pltpu.VMEM_SHARED((N_PAD, QUART), jnp.float32),
             pltpu.VMEM_SHARED((N_PAD, QUART), jnp.float32)]
          + [pltpu.SemaphoreType.DMA] * 18
      ),
      compiler_params=_SC_PARAMS,
  )
  return f(emb_st, src4, dst4, val4)


NB_SUB = N_BATCH // NS
NP_SUB = N_PAIRS // NS


def _gather_kernel(emb, g1l1, g1l2, g1l3, g2l1, g2l2, g2l3,
                   gfl1, gfl2, gfl3, nodes2, lists3,
                   e1s, e2s, us, vs, ns_, u0, v0, n0,
                   idxb, buf, sem):
  c = lax.axis_index("c")
  s = lax.axis_index("s")

  def gather_sum(tables, idx_hbm, nsubc, out, raw_out):
    pltpu.sync_copy(idx_hbm, idxb.at[pl.ds(0, nsubc)])
    n = nsubc * SUB
    for t, tbl in enumerate(tables):
      hs = []
      for j in range(nsubc):
        hs.append(pltpu.async_copy(
            tbl.at[c].at[idxb.at[j]],
            buf.at[pl.ds(j * SUB, SUB)], sem, add=(t > 0)))
      for h in hs:
        h.wait()
      if t == 0 and raw_out is not None:
        pltpu.sync_copy(buf.at[pl.ds(0, n)],
                        raw_out.at[c, pl.ds(s * n, n)])
    pltpu.sync_copy(buf.at[pl.ds(0, n)], out.at[c, pl.ds(s * n, n)])

  g1 = (emb, g1l1, g1l2, g1l3)
  g2 = (emb, g2l1, g2l2, g2l3)
  gf = (emb, gfl1, gfl2, gfl3)
  gather_sum(g1, nodes2.at[s], NB_SUB // SUB, e1s, None)
  gather_sum(g2, nodes2.at[s], NB_SUB // SUB, e2s, None)
  gather_sum(gf, lists3.at[0, s], NP_SUB // SUB, us, u0)
  gather_sum(gf, lists3.at[1, s], NP_SUB // SUB, vs, v0)
  gather_sum(gf, lists3.at[2, s], NP_SUB // SUB, ns_, n0)


@jax.jit
def _gather_combine(emb_st, g1o, g2o, gfo, nodes, node_list, pos_list,
                    neg_list):
  nodes2 = nodes.reshape(NS, NB_SUB // SUB, SUB)
  lists3 = jnp.stack([node_list, pos_list, neg_list]).reshape(
      3, NS, NP_SUB // SUB, SUB)
  f = pl.kernel(
      _gather_kernel,
      out_type=[jax.ShapeDtypeStruct((NC, N_BATCH, HALF), jnp.float32)] * 2
      + [jax.ShapeDtypeStruct((NC, N_PAIRS, HALF), jnp.float32)] * 6,
      mesh=_SC_MESH,
      scratch_types=[
          pltpu.VMEM((NP_SUB // SUB, SUB), jnp.int32),
          pltpu.VMEM((NP_SUB, HALF), jnp.float32),
          pltpu.SemaphoreType.DMA,
      ],
      compiler_params=_SC_PARAMS,
  )
  return f(emb_st, g1o[0], g1o[1], g1o[2], g2o[0], g2o[1], g2o[2],
           gfo[0], gfo[1], gfo[2], nodes2, lists3)


SSL_BLK = 512
SSL_STEPS = N_BATCH // SSL_BLK


def _l1n(x):
  return x / jnp.clip(jnp.sum(jnp.abs(x), axis=1, keepdims=True), 1e-12, None)


def _ssl_tc_kernel(e1_ref, e2_ref, out_ref):
  step = pl.program_id(0)
  n1 = _l1n(e1_ref[...])
  n2 = _l1n(e2_ref[...])
  n2_blk = _l1n(e2_ref[pl.ds(step * SSL_BLK, SSL_BLK), :])
  dots = jnp.sum(n1 * n2_blk, axis=1)
  s = lax.dot_general(n1, n2, (((1,), (1,)), ((), ())),
                      preferred_element_type=jnp.float32) / TEMP
  ttl = jnp.sum(jnp.exp(s), axis=1)
  partial = jnp.sum(jnp.log(ttl) - dots / TEMP)
  prev = jnp.where(step == 0, 0.0, out_ref[0])
  out_ref[0] = prev + partial


def _ssl_loss_tc(e1, e2):
  return pl.pallas_call(
      _ssl_tc_kernel,
      grid=(SSL_STEPS,),
      in_specs=[
          pl.BlockSpec((SSL_BLK, N_DIM), lambda i: (i, 0)),
          pl.BlockSpec((N_BATCH, N_DIM), lambda i: (0, 0)),
      ],
      out_specs=pl.BlockSpec(memory_space=pltpu.MemorySpace.SMEM),
      out_shape=jax.ShapeDtypeStruct((1,), jnp.float32),
  )(e1, e2)[0]


BPR_BLK = 1024
BPR_STEPS = N_PAIRS // BPR_BLK


def _bpr_tc_kernel(u_ref, v_ref, n_ref, u0_ref, v0_ref, n0_ref,
                   bpr_ref, reg_ref):
  step = pl.program_id(0)
  u = u_ref[...]
  pos = jnp.sum(u * v_ref[...], axis=1) / 16.0
  neg = jnp.sum(u * n_ref[...], axis=1) / 16.0
  d = neg - pos
  sp = jnp.log1p(jnp.exp(-jnp.abs(d))) + jnp.maximum(d, 0.0)
  reg = (jnp.sum(u0_ref[...] ** 2) + jnp.sum(v0_ref[...] ** 2)
         + jnp.sum(n0_ref[...] ** 2))
  prev_b = jnp.where(step == 0, 0.0, bpr_ref[0])
  prev_r = jnp.where(step == 0, 0.0, reg_ref[0])
  bpr_ref[0] = prev_b + jnp.sum(sp)
  reg_ref[0] = prev_r + reg


def _bpr_reg_tc(u, v, n, u0, v0, n0):
  spec = pl.BlockSpec((BPR_BLK, N_DIM), lambda i: (i, 0))
  return pl.pallas_call(
      _bpr_tc_kernel,
      grid=(BPR_STEPS,),
      in_specs=[spec] * 6,
      out_specs=[pl.BlockSpec(memory_space=pltpu.MemorySpace.SMEM)] * 2,
      out_shape=[jax.ShapeDtypeStruct((1,), jnp.float32)] * 2,
  )(u, v, n, u0, v0, n0)


def _unstack(x):
  return jnp.concatenate([x[0], x[1]], axis=1)


def kernel(training, graph1_index, graph1_values, graph2_index, graph2_values,
           graph_index, graph_values, nodes, node_list, pos_list, neg_list,
           embeddings):
  embp = jnp.concatenate(
      [embeddings, jnp.zeros((N_PAD - N_NODES, N_DIM), jnp.float32)])
  emb_st = jnp.stack([embp[:, :HALF], embp[:, HALF:]])

  g1 = _propagate3(emb_st, graph1_index, graph1_values)
  g2 = _propagate3(emb_st, graph2_index, graph2_values)
  gf = _propagate3(emb_st, graph_index, graph_values)

  e1s, e2s, us, vs, ns_, u0, v0, n0 = _gather_combine(
      emb_st, g1, g2, gf, nodes, node_list, pos_list, neg_list)

  ssl_loss = _ssl_loss_tc(_unstack(e1s), _unstack(e2s))
  bpr_sum, reg_sum = _bpr_reg_tc(
      _unstack(us), _unstack(vs), _unstack(ns_),
      _unstack(u0), _unstack(v0), _unstack(n0))
  bpr_loss = bpr_sum[0] / float(N_PAIRS)
  reg_loss = 0.5 * reg_sum[0] / float(N_BATCH)
  return (ssl_loss * LAMBDA_SSL + bpr_loss * LAMBDA_BPR
          + reg_loss * LAMBDA_REG)


# speedup vs baseline: 1.6635x; 1.6635x over previous
import jax
import jax.numpy as jnp
from jax import lax
from jax.experimental import pallas as pl
from jax.experimental.pallas import tpu as pltpu
from jax.experimental.pallas import tpu_sc as plsc

N_NODES = 50000
N_PAD = 50048
N_DIM = 64
HALF = 32
N_LAYERS = 3
N_BATCH = 4096
N_PAIRS = 16384
TEMP = 0.5
LAMBDA_SSL = 1.0
LAMBDA_BPR = 1.0
LAMBDA_REG = 1e-4

NC = 2
NS = 16
SUB = 128
NSUB = 1
CHUNK = SUB * NSUB
EDGE_SUB = 800000 // NS
NCHUNK = 390
NSETS = 6
NTRIP_IT = NCHUNK // NSETS
TAIL = EDGE_SUB - NCHUNK * SUB
ROWS_PER_SUB = N_PAD // NS
ZROWS = 96

_SC_MESH = plsc.VectorSubcoreMesh(core_axis_name="c", subcore_axis_name="s",
                                  num_cores=NC, num_subcores=NS)
_SC_PARAMS = pltpu.CompilerParams(use_tc_tiling_on_sc=False)


def _spmm3_kernel(emb, src2, dst2, val2, out1, out2, out3, *refs):
  srcvs = refs[0:6]
  dstvs = refs[6:12]
  valvs = refs[12:18]
  rowss = refs[18:24]
  srcvt = refs[24]
  dstvt = refs[25]
  valvt = refs[26]
  zbuf = refs[27]
  acc = refs[28]
  semis = refs[29:35]
  semgs = refs[35:41]
  semss = refs[41:47]
  c = lax.axis_index("c")
  s = lax.axis_index("s")

  def _z(i, _):
    zv = jnp.zeros((16,), jnp.float32)
    zbuf[i, pl.ds(0, 16)] = zv
    zbuf[i, pl.ds(16, 16)] = zv
    return 0
  lax.fori_loop(0, ZROWS, _z, 0)

  def issue_i(g, x):
    pltpu.async_copy(src2.at[s, pl.ds(g * SUB, SUB)], srcvs[x], semis[x])
    pltpu.async_copy(dst2.at[s, pl.ds(g * SUB, SUB)], dstvs[x], semis[x])
    pltpu.async_copy(val2.at[s, pl.ds(g * SUB, SUB)], valvs[x], semis[x])

  def wait_i(g, x):
    pltpu.make_async_copy(src2.at[s, pl.ds(g * SUB, SUB)], srcvs[x],
                          semis[x]).wait()
    pltpu.make_async_copy(dst2.at[s, pl.ds(g * SUB, SUB)], dstvs[x],
                          semis[x]).wait()
    pltpu.make_async_copy(val2.at[s, pl.ds(g * SUB, SUB)], valvs[x],
                          semis[x]).wait()

  def issue_g(tbl, x):
    pltpu.async_copy(tbl.at[c].at[srcvs[x]], rowss[x], semgs[x])

  def wait_g(tbl, x):
    pltpu.make_async_copy(tbl.at[c].at[srcvs[x]], rowss[x],
                          semgs[x]).wait()

  def issue_s(x):
    pltpu.async_copy(rowss[x], acc.at[dstvs[x]], semss[x], add=True)

  def wait_s(x):
    pltpu.make_async_copy(rowss[x], acc.at[dstvs[x]], semss[x]).wait()

  def do_scale_on(valv, rows, n16):
    def _scale(e0, _):
      vv = valv[pl.ds(e0 * 16, 16)]
      for l in range(16):
        v = vv[l]
        e = e0 * 16 + l
        rows[e, pl.ds(0, 16)] = rows[e, pl.ds(0, 16)] * v
        rows[e, pl.ds(16, 16)] = rows[e, pl.ds(16, 16)] * v
      return 0
    lax.fori_loop(0, n16, _scale, 0)

  def do_scale(x):
    do_scale_on(valvs[x], rowss[x], SUB // 16)

  srcs = (emb, out1, out2)
  outs = (out1, out2, out3)

  for lyr in range(N_LAYERS):
    tbl = srcs[lyr]
    out = outs[lyr]

    base = s * ROWS_PER_SUB
    for k in range(ROWS_PER_SUB // ZROWS):
      pltpu.sync_copy(zbuf, acc.at[pl.ds(base + k * ZROWS, ZROWS)])
    rem = ROWS_PER_SUB % ZROWS
    if rem:
      pltpu.sync_copy(zbuf.at[pl.ds(0, rem)],
                      acc.at[pl.ds(base + ROWS_PER_SUB - rem, rem)])
    plsc.subcore_barrier()

    issue_i(0, 0)
    issue_i(1, 1)
    issue_i(2, 2)
    wait_i(0, 0)
    issue_g(tbl, 0)
    wait_i(1, 1)
    issue_g(tbl, 1)

    def _trip(i, _):
      for x in range(NSETS):
        g = NSETS * i + x
        if x >= 3:
          wait_s((x + 3) % NSETS)
        else:
          @pl.when(i >= 1)
          def _():
            wait_s((x + 3) % NSETS)
        if x < 3:
          issue_i(g + 3, (x + 3) % NSETS)
        else:
          @pl.when(i < NTRIP_IT - 1)
          def _():
            issue_i(g + 3, (x + 3) % NSETS)
        if x < 4:
          wait_i(g + 2, (x + 2) % NSETS)
          issue_g(tbl, (x + 2) % NSETS)
        else:
          @pl.when(i < NTRIP_IT - 1)
          def _():
            wait_i(g + 2, (x + 2) % NSETS)
            issue_g(tbl, (x + 2) % NSETS)
        wait_g(tbl, x)
        do_scale(x)
        issue_s(x)
      return 0
    lax.fori_loop(0, NTRIP_IT, _trip, 0)
    wait_s(3)
    wait_s(4)
    wait_s(5)
    t0 = NCHUNK * SUB
    pltpu.sync_copy(src2.at[s, pl.ds(t0, TAIL)], srcvt)
    pltpu.sync_copy(dst2.at[s, pl.ds(t0, TAIL)], dstvt)
    pltpu.sync_copy(val2.at[s, pl.ds(t0, TAIL)], valvt)
    rt = rowss[0].at[pl.ds(0, TAIL)]
    pltpu.async_copy(tbl.at[c].at[srcvt], rt, semgs[0]).wait()
    do_scale_on(valvt, rowss[0], TAIL // 16)
    pltpu.async_copy(rt, acc.at[dstvt], semss[0], add=True).wait()
    plsc.subcore_barrier()

    pltpu.sync_copy(acc.at[pl.ds(base, ROWS_PER_SUB)],
                    out.at[c, pl.ds(base, ROWS_PER_SUB)])
    plsc.subcore_barrier()


@jax.jit
def _propagate3(emb_st, idx, vals):
  src2 = idx[1].reshape(NS, EDGE_SUB)
  dst2 = idx[0].reshape(NS, EDGE_SUB)
  val2 = vals.reshape(NS, EDGE_SUB)

  f = pl.kernel(
      _spmm3_kernel,
      out_type=[jax.ShapeDtypeStruct((NC, N_PAD, HALF), jnp.float32)] * 3,
      mesh=_SC_MESH,
      scratch_types=(
          [pltpu.VMEM((SUB,), jnp.int32)] * 12
          + [pltpu.VMEM((SUB,), jnp.float32)] * 6
          + [pltpu.VMEM((SUB, HALF), jnp.float32)] * 6
          + [pltpu.VMEM((TAIL,), jnp.int32)] * 2
          + [pltpu.VMEM((TAIL,), jnp.float32),
             pltpu.VMEM((ZROWS, HALF), jnp.float32),
             pltpu.VMEM_SHARED((N_PAD, HALF), jnp.float32)]
          + [pltpu.SemaphoreType.DMA] * 18
      ),
      compiler_params=_SC_PARAMS,
  )
  return f(emb_st, src2, dst2, val2)


NB_SUB = N_BATCH // NS
NP_SUB = N_PAIRS // NS


def _gather_kernel(emb, g1l1, g1l2, g1l3, g2l1, g2l2, g2l3,
                   gfl1, gfl2, gfl3, nodes2, lists3,
                   e1s, e2s, us, vs, ns_, u0, v0, n0,
                   idxb, buf, sem):
  c = lax.axis_index("c")
  s = lax.axis_index("s")

  def gather_sum(tables, idx_hbm, nsubc, out, raw_out):
    pltpu.sync_copy(idx_hbm, idxb.at[pl.ds(0, nsubc)])
    n = nsubc * SUB
    for t, tbl in enumerate(tables):
      hs = []
      for j in range(nsubc):
        hs.append(pltpu.async_copy(
            tbl.at[c].at[idxb.at[j]],
            buf.at[pl.ds(j * SUB, SUB)], sem, add=(t > 0)))
      for h in hs:
        h.wait()
      if t == 0 and raw_out is not None:
        pltpu.sync_copy(buf.at[pl.ds(0, n)],
                        raw_out.at[c, pl.ds(s * n, n)])
    pltpu.sync_copy(buf.at[pl.ds(0, n)], out.at[c, pl.ds(s * n, n)])

  g1 = (emb, g1l1, g1l2, g1l3)
  g2 = (emb, g2l1, g2l2, g2l3)
  gf = (emb, gfl1, gfl2, gfl3)
  gather_sum(g1, nodes2.at[s], NB_SUB // SUB, e1s, None)
  gather_sum(g2, nodes2.at[s], NB_SUB // SUB, e2s, None)
  gather_sum(gf, lists3.at[0, s], NP_SUB // SUB, us, u0)
  gather_sum(gf, lists3.at[1, s], NP_SUB // SUB, vs, v0)
  gather_sum(gf, lists3.at[2, s], NP_SUB // SUB, ns_, n0)


@jax.jit
def _gather_combine(emb_st, g1o, g2o, gfo, nodes, node_list, pos_list,
                    neg_list):
  nodes2 = nodes.reshape(NS, NB_SUB // SUB, SUB)
  lists3 = jnp.stack([node_list, pos_list, neg_list]).reshape(
      3, NS, NP_SUB // SUB, SUB)
  f = pl.kernel(
      _gather_kernel,
      out_type=[jax.ShapeDtypeStruct((NC, N_BATCH, HALF), jnp.float32)] * 2
      + [jax.ShapeDtypeStruct((NC, N_PAIRS, HALF), jnp.float32)] * 6,
      mesh=_SC_MESH,
      scratch_types=[
          pltpu.VMEM((NP_SUB // SUB, SUB), jnp.int32),
          pltpu.VMEM((NP_SUB, HALF), jnp.float32),
          pltpu.SemaphoreType.DMA,
      ],
      compiler_params=_SC_PARAMS,
  )
  return f(emb_st, g1o[0], g1o[1], g1o[2], g2o[0], g2o[1], g2o[2],
           gfo[0], gfo[1], gfo[2], nodes2, lists3)


SSL_BLK = 512
SSL_STEPS = N_BATCH // SSL_BLK


def _l1n(x):
  return x / jnp.clip(jnp.sum(jnp.abs(x), axis=1, keepdims=True), 1e-12, None)


def _ssl_tc_kernel(e1_ref, e2_ref, out_ref):
  step = pl.program_id(0)
  n1 = _l1n(e1_ref[...])
  n2 = _l1n(e2_ref[...])
  n2_blk = _l1n(e2_ref[pl.ds(step * SSL_BLK, SSL_BLK), :])
  dots = jnp.sum(n1 * n2_blk, axis=1)
  s = lax.dot_general(n1, n2, (((1,), (1,)), ((), ())),
                      preferred_element_type=jnp.float32) / TEMP
  ttl = jnp.sum(jnp.exp(s), axis=1)
  partial = jnp.sum(jnp.log(ttl) - dots / TEMP)
  prev = jnp.where(step == 0, 0.0, out_ref[0])
  out_ref[0] = prev + partial


def _ssl_loss_tc(e1, e2):
  return pl.pallas_call(
      _ssl_tc_kernel,
      grid=(SSL_STEPS,),
      in_specs=[
          pl.BlockSpec((SSL_BLK, N_DIM), lambda i: (i, 0)),
          pl.BlockSpec((N_BATCH, N_DIM), lambda i: (0, 0)),
      ],
      out_specs=pl.BlockSpec(memory_space=pltpu.MemorySpace.SMEM),
      out_shape=jax.ShapeDtypeStruct((1,), jnp.float32),
  )(e1, e2)[0]


BPR_BLK = 1024
BPR_STEPS = N_PAIRS // BPR_BLK


def _bpr_tc_kernel(u_ref, v_ref, n_ref, u0_ref, v0_ref, n0_ref,
                   bpr_ref, reg_ref):
  step = pl.program_id(0)
  u = u_ref[...]
  pos = jnp.sum(u * v_ref[...], axis=1) / 16.0
  neg = jnp.sum(u * n_ref[...], axis=1) / 16.0
  d = neg - pos
  sp = jnp.log1p(jnp.exp(-jnp.abs(d))) + jnp.maximum(d, 0.0)
  reg = (jnp.sum(u0_ref[...] ** 2) + jnp.sum(v0_ref[...] ** 2)
         + jnp.sum(n0_ref[...] ** 2))
  prev_b = jnp.where(step == 0, 0.0, bpr_ref[0])
  prev_r = jnp.where(step == 0, 0.0, reg_ref[0])
  bpr_ref[0] = prev_b + jnp.sum(sp)
  reg_ref[0] = prev_r + reg


def _bpr_reg_tc(u, v, n, u0, v0, n0):
  spec = pl.BlockSpec((BPR_BLK, N_DIM), lambda i: (i, 0))
  return pl.pallas_call(
      _bpr_tc_kernel,
      grid=(BPR_STEPS,),
      in_specs=[spec] * 6,
      out_specs=[pl.BlockSpec(memory_space=pltpu.MemorySpace.SMEM)] * 2,
      out_shape=[jax.ShapeDtypeStruct((1,), jnp.float32)] * 2,
  )(u, v, n, u0, v0, n0)


def _unstack(x):
  return jnp.concatenate([x[0], x[1]], axis=1)


def kernel(training, graph1_index, graph1_values, graph2_index, graph2_values,
           graph_index, graph_values, nodes, node_list, pos_list, neg_list,
           embeddings):
  emb_st = jnp.stack([embeddings[:, :HALF], embeddings[:, HALF:]])

  g1 = _propagate3(emb_st, graph1_index, graph1_values)
  g2 = _propagate3(emb_st, graph2_index, graph2_values)
  gf = _propagate3(emb_st, graph_index, graph_values)

  e1s, e2s, us, vs, ns_, u0, v0, n0 = _gather_combine(
      emb_st, g1, g2, gf, nodes, node_list, pos_list, neg_list)

  ssl_loss = _ssl_loss_tc(_unstack(e1s), _unstack(e2s))
  bpr_sum, reg_sum = _bpr_reg_tc(
      _unstack(us), _unstack(vs), _unstack(ns_),
      _unstack(u0), _unstack(v0), _unstack(n0))
  bpr_loss = bpr_sum[0] / float(N_PAIRS)
  reg_loss = 0.5 * reg_sum[0] / float(N_BATCH)
  return (ssl_loss * LAMBDA_SSL + bpr_loss * LAMBDA_BPR
          + reg_loss * LAMBDA_REG)


# speedup vs baseline: 1.7510x; 1.0526x over previous
import jax
import jax.numpy as jnp
from jax import lax
from jax.experimental import pallas as pl
from jax.experimental.pallas import tpu as pltpu
from jax.experimental.pallas import tpu_sc as plsc

N_NODES = 50000
N_PAD = 50048
N_DIM = 64
HALF = 32
N_LAYERS = 3
N_BATCH = 4096
N_PAIRS = 16384
TEMP = 0.5
LAMBDA_SSL = 1.0
LAMBDA_BPR = 1.0
LAMBDA_REG = 1e-4

NC = 2
NS = 16
SUB = 128
NSUB = 1
CHUNK = SUB * NSUB
EDGE_SUB = 800000 // NS
NCHUNK = 390
NSETS = 6
NTRIP_IT = NCHUNK // NSETS
TAIL = EDGE_SUB - NCHUNK * SUB
ROWS_PER_SUB = N_PAD // NS
ZROWS = 96

_SC_MESH = plsc.VectorSubcoreMesh(core_axis_name="c", subcore_axis_name="s",
                                  num_cores=NC, num_subcores=NS)
_SC_PARAMS = pltpu.CompilerParams(use_tc_tiling_on_sc=False)


def _spmm3_kernel(emb, src2, dst2, val2, out1, out2, out3, *refs):
  srcvs = refs[0:6]
  dstvs = refs[6:12]
  valvs = refs[12:18]
  rowss = refs[18:24]
  srcvt = refs[24]
  dstvt = refs[25]
  valvt = refs[26]
  zbuf = refs[27]
  acc = refs[28]
  semis = refs[29:35]
  semgs = refs[35:41]
  semss = refs[41:47]
  c = lax.axis_index("c")
  s = lax.axis_index("s")

  def _z(i, _):
    zv = jnp.zeros((16,), jnp.float32)
    zbuf[i, pl.ds(0, 16)] = zv
    zbuf[i, pl.ds(16, 16)] = zv
    return 0
  lax.fori_loop(0, ZROWS, _z, 0)

  def issue_i(g, x):
    pltpu.async_copy(src2.at[s, pl.ds(g * SUB, SUB)], srcvs[x], semis[x])
    pltpu.async_copy(dst2.at[s, pl.ds(g * SUB, SUB)], dstvs[x], semis[x])
    pltpu.async_copy(val2.at[s, pl.ds(g * SUB, SUB)], valvs[x], semis[x])

  def wait_i(g, x):
    pltpu.make_async_copy(src2.at[s, pl.ds(g * SUB, SUB)], srcvs[x],
                          semis[x]).wait()
    pltpu.make_async_copy(dst2.at[s, pl.ds(g * SUB, SUB)], dstvs[x],
                          semis[x]).wait()
    pltpu.make_async_copy(val2.at[s, pl.ds(g * SUB, SUB)], valvs[x],
                          semis[x]).wait()

  def issue_g(tbl, x):
    pltpu.async_copy(tbl.at[c].at[srcvs[x]], rowss[x], semgs[x])

  def wait_g(tbl, x):
    pltpu.make_async_copy(tbl.at[c].at[srcvs[x]], rowss[x],
                          semgs[x]).wait()

  def issue_s(x):
    pltpu.async_copy(rowss[x], acc.at[dstvs[x]], semss[x], add=True)

  def wait_s(x):
    pltpu.make_async_copy(rowss[x], acc.at[dstvs[x]], semss[x]).wait()

  def do_scale_on(valv, rows, n16):
    def _scale(e0, _):
      vv = valv[pl.ds(e0 * 16, 16)]
      for l in range(16):
        v = vv[l]
        e = e0 * 16 + l
        rows[e, pl.ds(0, 16)] = rows[e, pl.ds(0, 16)] * v
        rows[e, pl.ds(16, 16)] = rows[e, pl.ds(16, 16)] * v
      return 0
    lax.fori_loop(0, n16, _scale, 0)

  def do_scale(x):
    do_scale_on(valvs[x], rowss[x], SUB // 16)

  srcs = (emb, out1, out2)
  outs = (out1, out2, out3)

  for lyr in range(N_LAYERS):
    tbl = srcs[lyr]
    out = outs[lyr]

    base = s * ROWS_PER_SUB
    for k in range(ROWS_PER_SUB // ZROWS):
      pltpu.sync_copy(zbuf, acc.at[pl.ds(base + k * ZROWS, ZROWS)])
    rem = ROWS_PER_SUB % ZROWS
    if rem:
      pltpu.sync_copy(zbuf.at[pl.ds(0, rem)],
                      acc.at[pl.ds(base + ROWS_PER_SUB - rem, rem)])
    plsc.subcore_barrier()

    issue_i(0, 0)
    issue_i(1, 1)
    issue_i(2, 2)
    wait_i(0, 0)
    issue_g(tbl, 0)
    wait_i(1, 1)
    issue_g(tbl, 1)

    def _trip(i, _):
      for x in range(NSETS):
        g = NSETS * i + x
        if x >= 3:
          wait_s((x + 3) % NSETS)
        else:
          @pl.when(i >= 1)
          def _():
            wait_s((x + 3) % NSETS)
        if x < 3:
          issue_i(g + 3, (x + 3) % NSETS)
        else:
          @pl.when(i < NTRIP_IT - 1)
          def _():
            issue_i(g + 3, (x + 3) % NSETS)
        if x < 4:
          wait_i(g + 2, (x + 2) % NSETS)
          issue_g(tbl, (x + 2) % NSETS)
        else:
          @pl.when(i < NTRIP_IT - 1)
          def _():
            wait_i(g + 2, (x + 2) % NSETS)
            issue_g(tbl, (x + 2) % NSETS)
        wait_g(tbl, x)
        do_scale(x)
        issue_s(x)
      return 0
    lax.fori_loop(0, NTRIP_IT, _trip, 0)
    wait_s(3)
    wait_s(4)
    wait_s(5)
    t0 = NCHUNK * SUB
    pltpu.sync_copy(src2.at[s, pl.ds(t0, TAIL)], srcvt)
    pltpu.sync_copy(dst2.at[s, pl.ds(t0, TAIL)], dstvt)
    pltpu.sync_copy(val2.at[s, pl.ds(t0, TAIL)], valvt)
    rt = rowss[0].at[pl.ds(0, TAIL)]
    pltpu.async_copy(tbl.at[c].at[srcvt], rt, semgs[0]).wait()
    do_scale_on(valvt, rowss[0], TAIL // 16)
    pltpu.async_copy(rt, acc.at[dstvt], semss[0], add=True).wait()
    plsc.subcore_barrier()

    pltpu.sync_copy(acc.at[pl.ds(base, ROWS_PER_SUB)],
                    out.at[c, pl.ds(base, ROWS_PER_SUB)])
    plsc.subcore_barrier()


@jax.jit
def _propagate3(emb_st, idx, vals):
  src2 = idx[1].reshape(NS, EDGE_SUB)
  dst2 = idx[0].reshape(NS, EDGE_SUB)
  val2 = vals.reshape(NS, EDGE_SUB)

  f = pl.kernel(
      _spmm3_kernel,
      out_type=[jax.ShapeDtypeStruct((NC, N_PAD, HALF), jnp.float32)] * 3,
      mesh=_SC_MESH,
      scratch_types=(
          [pltpu.VMEM((SUB,), jnp.int32)] * 12
          + [pltpu.VMEM((SUB,), jnp.float32)] * 6
          + [pltpu.VMEM((SUB, HALF), jnp.float32)] * 6
          + [pltpu.VMEM((TAIL,), jnp.int32)] * 2
          + [pltpu.VMEM((TAIL,), jnp.float32),
             pltpu.VMEM((ZROWS, HALF), jnp.float32),
             pltpu.VMEM_SHARED((N_PAD, HALF), jnp.float32)]
          + [pltpu.SemaphoreType.DMA] * 18
      ),
      compiler_params=_SC_PARAMS,
  )
  return f(emb_st, src2, dst2, val2)


NB_SUB = N_BATCH // NS
NP_SUB = N_PAIRS // NS


def _gather_kernel(emb, g1l1, g1l2, g1l3, g2l1, g2l2, g2l3,
                   gfl1, gfl2, gfl3, nodes2, lists3,
                   e1s, e2s, us, vs, ns_, u0, v0, n0,
                   idxb, buf, sem):
  c = lax.axis_index("c")
  s = lax.axis_index("s")

  def gather_sum(tables, idx_hbm, nsubc, out, raw_out):
    pltpu.sync_copy(idx_hbm, idxb.at[pl.ds(0, nsubc)])
    n = nsubc * SUB
    for t, tbl in enumerate(tables):
      hs = []
      for j in range(nsubc):
        hs.append(pltpu.async_copy(
            tbl.at[c].at[idxb.at[j]],
            buf.at[pl.ds(j * SUB, SUB)], sem, add=(t > 0)))
      for h in hs:
        h.wait()
      if t == 0 and raw_out is not None:
        pltpu.sync_copy(buf.at[pl.ds(0, n)],
                        raw_out.at[c, pl.ds(s * n, n)])
    pltpu.sync_copy(buf.at[pl.ds(0, n)], out.at[c, pl.ds(s * n, n)])

  g1 = (emb, g1l1, g1l2, g1l3)
  g2 = (emb, g2l1, g2l2, g2l3)
  gf = (emb, gfl1, gfl2, gfl3)
  gather_sum(g1, nodes2.at[s], NB_SUB // SUB, e1s, None)
  gather_sum(g2, nodes2.at[s], NB_SUB // SUB, e2s, None)
  gather_sum(gf, lists3.at[0, s], NP_SUB // SUB, us, u0)
  gather_sum(gf, lists3.at[1, s], NP_SUB // SUB, vs, v0)
  gather_sum(gf, lists3.at[2, s], NP_SUB // SUB, ns_, n0)


@jax.jit
def _gather_combine(emb_st, g1o, g2o, gfo, nodes, node_list, pos_list,
                    neg_list):
  nodes2 = nodes.reshape(NS, NB_SUB // SUB, SUB)
  lists3 = jnp.stack([node_list, pos_list, neg_list]).reshape(
      3, NS, NP_SUB // SUB, SUB)
  f = pl.kernel(
      _gather_kernel,
      out_type=[jax.ShapeDtypeStruct((NC, N_BATCH, HALF), jnp.float32)] * 2
      + [jax.ShapeDtypeStruct((NC, N_PAIRS, HALF), jnp.float32)] * 6,
      mesh=_SC_MESH,
      scratch_types=[
          pltpu.VMEM((NP_SUB // SUB, SUB), jnp.int32),
          pltpu.VMEM((NP_SUB, HALF), jnp.float32),
          pltpu.SemaphoreType.DMA,
      ],
      compiler_params=_SC_PARAMS,
  )
  return f(emb_st, g1o[0], g1o[1], g1o[2], g2o[0], g2o[1], g2o[2],
           gfo[0], gfo[1], gfo[2], nodes2, lists3)


SSL_BLK = 512
SSL_STEPS = N_BATCH // SSL_BLK


def _l1n(x):
  return x / jnp.clip(jnp.sum(jnp.abs(x), axis=1, keepdims=True), 1e-12, None)


def _ssl_tc_kernel(e1_ref, e2_ref, out_ref):
  step = pl.program_id(0)
  a = e1_ref[...]
  b = e2_ref[...]
  d1 = jnp.clip(jnp.sum(jnp.abs(a), axis=(0, 2), keepdims=True), 1e-12, None)
  n1 = a / d1
  d2 = jnp.clip(jnp.sum(jnp.abs(b), axis=(0, 2), keepdims=True), 1e-12, None)
  n2 = b / d2
  bb = e2_ref[:, pl.ds(step * SSL_BLK, SSL_BLK), :]
  db = jnp.clip(jnp.sum(jnp.abs(bb), axis=(0, 2), keepdims=True), 1e-12, None)
  dots = jnp.sum(n1 * (bb / db), axis=(0, 2))
  dn = (((2,), (2,)), ((0,), (0,)))
  s = (lax.dot_general(n1[:1], n2[:1], dn,
                       preferred_element_type=jnp.float32)[0]
       + lax.dot_general(n1[1:], n2[1:], dn,
                         preferred_element_type=jnp.float32)[0]) / TEMP
  ttl = jnp.sum(jnp.exp(s), axis=1)
  partial = jnp.sum(jnp.log(ttl) - dots / TEMP)
  prev = jnp.where(step == 0, 0.0, out_ref[0])
  out_ref[0] = prev + partial


def _ssl_loss_tc(e1, e2):
  return pl.pallas_call(
      _ssl_tc_kernel,
      grid=(SSL_STEPS,),
      in_specs=[
          pl.BlockSpec((2, SSL_BLK, HALF), lambda i: (0, i, 0)),
          pl.BlockSpec((2, N_BATCH, HALF), lambda i: (0, 0, 0)),
      ],
      out_specs=pl.BlockSpec(memory_space=pltpu.MemorySpace.SMEM),
      out_shape=jax.ShapeDtypeStruct((1,), jnp.float32),
  )(e1, e2)[0]


BPR_BLK = 1024
BPR_STEPS = N_PAIRS // BPR_BLK


def _bpr_tc_kernel(u_ref, v_ref, n_ref, u0_ref, v0_ref, n0_ref,
                   bpr_ref, reg_ref):
  step = pl.program_id(0)
  u = u_ref[...]
  pos = jnp.sum(u * v_ref[...], axis=(0, 2)) / 16.0
  neg = jnp.sum(u * n_ref[...], axis=(0, 2)) / 16.0
  d = neg - pos
  sp = jnp.log1p(jnp.exp(-jnp.abs(d))) + jnp.maximum(d, 0.0)
  reg = (jnp.sum(u0_ref[...] ** 2) + jnp.sum(v0_ref[...] ** 2)
         + jnp.sum(n0_ref[...] ** 2))
  prev_b = jnp.where(step == 0, 0.0, bpr_ref[0])
  prev_r = jnp.where(step == 0, 0.0, reg_ref[0])
  bpr_ref[0] = prev_b + jnp.sum(sp)
  reg_ref[0] = prev_r + reg


def _bpr_reg_tc(u, v, n, u0, v0, n0):
  spec = pl.BlockSpec((2, BPR_BLK, HALF), lambda i: (0, i, 0))
  return pl.pallas_call(
      _bpr_tc_kernel,
      grid=(BPR_STEPS,),
      in_specs=[spec] * 6,
      out_specs=[pl.BlockSpec(memory_space=pltpu.MemorySpace.SMEM)] * 2,
      out_shape=[jax.ShapeDtypeStruct((1,), jnp.float32)] * 2,
  )(u, v, n, u0, v0, n0)


def kernel(training, graph1_index, graph1_values, graph2_index, graph2_values,
           graph_index, graph_values, nodes, node_list, pos_list, neg_list,
           embeddings):
  emb_st = jnp.stack([embeddings[:, :HALF], embeddings[:, HALF:]])

  g1 = _propagate3(emb_st, graph1_index, graph1_values)
  g2 = _propagate3(emb_st, graph2_index, graph2_values)
  gf = _propagate3(emb_st, graph_index, graph_values)

  e1s, e2s, us, vs, ns_, u0, v0, n0 = _gather_combine(
      emb_st, g1, g2, gf, nodes, node_list, pos_list, neg_list)

  ssl_loss = _ssl_loss_tc(e1s, e2s)
  bpr_sum, reg_sum = _bpr_reg_tc(us, vs, ns_, u0, v0, n0)
  bpr_loss = bpr_sum[0] / float(N_PAIRS)
  reg_loss = 0.5 * reg_sum[0] / float(N_BATCH)
  return (ssl_loss * LAMBDA_SSL + bpr_loss * LAMBDA_BPR
          + reg_loss * LAMBDA_REG)
